# baseline (speedup 1.0000x reference)
import functools

import jax
import jax.numpy as jnp
from jax import lax
from jax.experimental import pallas as pl
from jax.experimental.pallas import tpu as pltpu
from jax.experimental.pallas import tpu_sc as plsc

N_USER = 5000
N = 10000
D = 128
E = 320000
NUM_LAYERS = 3

NC, NS = 2, 16
NW = NC * NS
W = 128
NWIN = 80
NSUB = 4
SUBW = W // NSUB
EW_PAD = NWIN * W
E_PAD = NW * EW_PAD
N_ACC = 10240
DEG_W = 128
ROWS_PER_SUB = N_ACC // NS

_mesh = lambda: plsc.VectorSubcoreMesh(core_axis_name="c", subcore_axis_name="s")


def _fill_buf(buf, rows, width, value):
    chunks = width // 16

    def body(r, _):
        for c in range(chunks):
            buf[r, pl.ds(c * 16, 16)] = jnp.full((16,), value, jnp.float32)
        return 0

    lax.fori_loop(0, rows, body, 0)


def _zero_acc(acc, gbuf, sid):
    base = sid * ROWS_PER_SUB
    for k in range(ROWS_PER_SUB // W):
        pltpu.sync_copy(gbuf, acc.at[pl.ds(base + k * W, W)])


def _copy_out(acc, out_hbm, cid, sid):
    base = sid * ROWS_PER_SUB
    pltpu.sync_copy(acc.at[pl.ds(base, ROWS_PER_SUB)],
                    out_hbm.at[cid, pl.ds(base, ROWS_PER_SUB)])


def _unpack_win(packed_v, j, row_win, col_win):
    for c in range(W // 16):
        v = packed_v[j, pl.ds(c * 16, 16)]
        if row_win is not None:
            row_win[0, pl.ds(c * 16, 16)] = lax.shift_right_logical(v, 14)
        col_win[0, pl.ds(c * 16, 16)] = lax.bitwise_and(
            v, jnp.full((16,), 16383, jnp.int32))


def _make_scatter_kernel(gather):
    if gather:
        @functools.partial(
            pl.kernel,
            out_type=jax.ShapeDtypeStruct((NC, N_ACC, D), jnp.float32),
            mesh=_mesh(),
            scratch_types=[
                pltpu.VMEM((NWIN, W), jnp.int32),
                pltpu.VMEM((1, W), jnp.int32),
                pltpu.VMEM((1, W), jnp.int32),
                pltpu.VMEM((1, W), jnp.int32),
                pltpu.VMEM((1, W), jnp.int32),
                pltpu.VMEM((W, D), jnp.float32),
                pltpu.VMEM((W, D), jnp.float32),
                pltpu.VMEM_SHARED((N_ACC, D), jnp.float32),
                pltpu.SemaphoreType.DMA,
                pltpu.SemaphoreType.DMA,
            ],
        )
        def prop_kernel(y_hbm, packed_hbm, out_hbm,
                        packed_v, row0, row1, col0, col1, g0, g1, acc,
                        sem0, sem1):
            cid = lax.axis_index("c")
            sid = lax.axis_index("s")
            wid = cid * NS + sid
            pltpu.sync_copy(packed_hbm.at[wid], packed_v)
            _fill_buf(g0, W, D, 0.0)
            _zero_acc(acc, g0, sid)
            plsc.subcore_barrier()

            def gsub(row_win, buf, sem, q):
                return pltpu.make_async_copy(
                    y_hbm.at[row_win.at[0, pl.ds(q * SUBW, SUBW)]],
                    buf.at[pl.ds(q * SUBW, SUBW)], sem)

            def gstart(row_win, buf, sem):
                for q in range(NSUB):
                    gsub(row_win, buf, sem, q).start()

            def gwait(row_win, buf, sem):
                for q in range(NSUB):
                    gsub(row_win, buf, sem, q).wait()

            _unpack_win(packed_v, 0, row0, col0)
            _unpack_win(packed_v, 1, row1, col1)
            gstart(row0, g0, sem0)
            gstart(row1, g1, sem1)

            def body(t, _):
                j0 = 2 * t
                gwait(row0, g0, sem0)
                pltpu.sync_copy(g0, acc.at[col0.at[0]], add=True)

                @pl.when(t < NWIN // 2 - 1)
                def _():
                    _unpack_win(packed_v, j0 + 2, row0, col0)
                    gstart(row0, g0, sem0)

                gwait(row1, g1, sem1)
                pltpu.sync_copy(g1, acc.at[col1.at[0]], add=True)

                @pl.when(t < NWIN // 2 - 1)
                def _():
                    _unpack_win(packed_v, j0 + 3, row1, col1)
                    gstart(row1, g1, sem1)

                return 0

            lax.fori_loop(0, NWIN // 2, body, 0)
            plsc.subcore_barrier()
            _copy_out(acc, out_hbm, cid, sid)

        return prop_kernel

    @functools.partial(
        pl.kernel,
        out_type=jax.ShapeDtypeStruct((NC, N_ACC, DEG_W), jnp.float32),
        mesh=_mesh(),
        scratch_types=[
            pltpu.VMEM((NWIN, W), jnp.int32),
            pltpu.VMEM((1, W), jnp.int32),
            pltpu.VMEM((W, DEG_W), jnp.float32),
            pltpu.VMEM_SHARED((N_ACC, DEG_W), jnp.float32),
        ],
    )
    def deg_kernel(packed_hbm, out_hbm, packed_v, col0, gbuf, acc):
        cid = lax.axis_index("c")
        sid = lax.axis_index("s")
        wid = cid * NS + sid
        pltpu.sync_copy(packed_hbm.at[wid], packed_v)
        _fill_buf(gbuf, W, DEG_W, 0.0)
        base = sid * ROWS_PER_SUB
        for k in range(ROWS_PER_SUB // W):
            pltpu.sync_copy(gbuf, acc.at[pl.ds(base + k * W, W)])
        _fill_buf(gbuf, W, DEG_W, 1.0)
        plsc.subcore_barrier()

        def body(j, _):
            _unpack_win(packed_v, j, None, col0)
            pltpu.sync_copy(gbuf, acc.at[col0.at[0]], add=True)
            return 0

        lax.fori_loop(0, NWIN, body, 0)
        plsc.subcore_barrier()
        _copy_out(acc, out_hbm, cid, sid)

    return deg_kernel


_BN = 2000


def _tc_dinv_y0(deg2, emb):
    def body(deg_ref, emb_ref, dinv_ref, y0_ref):
        d = deg_ref[0, :, 0:1] + deg_ref[1, :, 0:1]
        dv = jnp.where(d > 0, lax.rsqrt(jnp.maximum(d, 1.0)), 0.0)
        dinv_ref[...] = dv
        y0_ref[...] = emb_ref[...] * dv

    return pl.pallas_call(
        body,
        grid=(N // _BN,),
        in_specs=[
            pl.BlockSpec((NC, _BN, DEG_W), lambda i: (0, i, 0)),
            pl.BlockSpec((_BN, D), lambda i: (i, 0)),
        ],
        out_specs=[
            pl.BlockSpec((_BN, 1), lambda i: (i, 0)),
            pl.BlockSpec((_BN, D), lambda i: (i, 0)),
        ],
        out_shape=[
            jax.ShapeDtypeStruct((N, 1), jnp.float32),
            jax.ShapeDtypeStruct((N, D), jnp.float32),
        ],
    )(deg2, emb)


def _tc_layer(acc2, dinv, xout, last):
    def body_mid(acc_ref, dinv_ref, xout_ref, y_ref, xo_ref):
        dv = dinv_ref[...]
        x = (acc_ref[0] + acc_ref[1]) * dv
        xo_ref[...] = xout_ref[...] + x
        y_ref[...] = x * dv

    def body_last(acc_ref, dinv_ref, xout_ref, out_ref):
        dv = dinv_ref[...]
        x = (acc_ref[0] + acc_ref[1]) * dv
        out_ref[...] = (xout_ref[...] + x) * (1.0 / (1 + NUM_LAYERS))

    in_specs = [
        pl.BlockSpec((NC, _BN, D), lambda i: (0, i, 0)),
        pl.BlockSpec((_BN, 1), lambda i: (i, 0)),
        pl.BlockSpec((_BN, D), lambda i: (i, 0)),
    ]
    if last:
        return pl.pallas_call(
            body_last,
            grid=(N // _BN,),
            in_specs=in_specs,
            out_specs=pl.BlockSpec((_BN, D), lambda i: (i, 0)),
            out_shape=jax.ShapeDtypeStruct((N, D), jnp.float32),
        )(acc2, dinv, xout)
    return pl.pallas_call(
        body_mid,
        grid=(N // _BN,),
        in_specs=in_specs,
        out_specs=[
            pl.BlockSpec((_BN, D), lambda i: (i, 0)),
            pl.BlockSpec((_BN, D), lambda i: (i, 0)),
        ],
        out_shape=[
            jax.ShapeDtypeStruct((N, D), jnp.float32),
            jax.ShapeDtypeStruct((N, D), jnp.float32),
        ],
    )(acc2, dinv, xout)


def kernel(edge_index, emb):
    row = edge_index[0]
    col = edge_index[1]
    pad = E_PAD - E
    prow = (jnp.arange(pad, dtype=jnp.int32) * 131) % N
    pcol = N + (jnp.arange(pad, dtype=jnp.int32) % (N_ACC - N))
    rowp = jnp.concatenate([row, prow])
    colp = jnp.concatenate([col, pcol])
    packed = (rowp * 16384 + colp).reshape(NW, NWIN, W)

    deg2 = _make_scatter_kernel(gather=False)(packed)
    dinv, y = _tc_dinv_y0(deg2, emb)

    prop = _make_scatter_kernel(gather=True)
    xout = emb
    final = None
    for l in range(NUM_LAYERS):
        acc2 = prop(y, packed)
        if l < NUM_LAYERS - 1:
            y, xout = _tc_layer(acc2, dinv, xout, last=False)
        else:
            final = _tc_layer(acc2, dinv, xout, last=True)
    return final[:N_USER], final[N_USER:]
